# baseline (speedup 1.0000x reference)
import functools

import jax
import jax.numpy as jnp
from jax import lax
from jax.experimental import pallas as pl
from jax.experimental.pallas import tpu as pltpu
from jax.experimental.pallas import tpu_sc as plsc

R, S = 4096, 200
D = 32
B = R * S
NC, NS = 2, 16
NW = NC * NS
BPW = B // NW
CHUNK = 1600
NCHUNK = BPW // CHUNK

_mesh = plsc.VectorSubcoreMesh(core_axis_name="c", subcore_axis_name="s")


@functools.partial(
    pl.kernel,
    out_type=jax.ShapeDtypeStruct((B, D), jnp.float32),
    mesh=_mesh,
    scratch_types=[
        pltpu.VMEM((CHUNK,), jnp.int32),
        pltpu.VMEM((CHUNK,), jnp.int32),
        pltpu.VMEM((CHUNK, D), jnp.float32),
        pltpu.VMEM((CHUNK, D), jnp.float32),
        pltpu.SemaphoreType.DMA,
        pltpu.SemaphoreType.DMA,
        pltpu.SemaphoreType.DMA,
        pltpu.SemaphoreType.DMA,
    ],
    compiler_params=pltpu.CompilerParams(use_tc_tiling_on_sc=False),
)
def _gather_kernel(idx_hbm, table_hbm, out_hbm,
                   idx0, idx1, rows0, rows1, g0, g1, w0, w1):
    wid = lax.axis_index("s") * NC + lax.axis_index("c")
    base = wid * BPW

    idx_v = (idx0, idx1)
    rows_v = (rows0, rows1)
    gsem = (g0, g1)
    wsem = (w0, w1)

    gd = [None] * NCHUNK
    wd = [None] * NCHUNK
    for c in range(NCHUNK):
        b = c & 1
        if c >= 2:
            wd[c - 2].wait()
        off = base + c * CHUNK
        pltpu.sync_copy(idx_hbm.at[pl.ds(off, CHUNK)], idx_v[b])
        gd[c] = pltpu.async_copy(table_hbm.at[idx_v[b]], rows_v[b], gsem[b])
        if c >= 1:
            pb = (c - 1) & 1
            gd[c - 1].wait()
            poff = base + (c - 1) * CHUNK
            wd[c - 1] = pltpu.async_copy(
                rows_v[pb], out_hbm.at[pl.ds(poff, CHUNK)], wsem[pb])
    last = NCHUNK - 1
    gd[last].wait()
    wd[last] = pltpu.async_copy(
        rows_v[last & 1], out_hbm.at[pl.ds(base + last * CHUNK, CHUNK)],
        wsem[last & 1])
    wd[last - 1].wait()
    wd[last].wait()


def kernel(words, table):
    idx = words.reshape(B).astype(jnp.int32)
    out = _gather_kernel(idx, table)
    return out.reshape(R, S, D)
